# baseline (speedup 1.0000x reference)
import functools

import jax
import jax.numpy as jnp
from jax import lax
from jax.experimental import pallas as pl
from jax.experimental.pallas import tpu as pltpu
from jax.experimental.pallas import tpu_sc as plsc

DIM = 64
NC = 2
NS = 16
NW = NC * NS

CPT = 128


@functools.lru_cache(maxsize=None)
def _make_kernel(T: int, B: int, V2: int):
    assert B == NW * CPT and T % 2 == 0
    mesh = plsc.VectorSubcoreMesh(core_axis_name="c", subcore_axis_name="s")

    @functools.partial(
        pl.kernel,
        mesh=mesh,
        compiler_params=pltpu.CompilerParams(
            use_tc_tiling_on_sc=True, needs_layout_passes=False),
        out_type=jax.ShapeDtypeStruct((T, DIM, B), jnp.float32),
        scratch_types=[
            pltpu.VMEM((T, CPT), jnp.int32),
            pltpu.VMEM((2, CPT), jnp.int32),
            pltpu.VMEM((2, CPT), jnp.int32),
            pltpu.VMEM((2, CPT, 2 * DIM), jnp.float32),
            pltpu.VMEM((2, DIM, CPT), jnp.float32),
            pltpu.SemaphoreType.DMA,
            pltpu.SemaphoreType.DMA,
            pltpu.SemaphoreType.DMA,
            pltpu.SemaphoreType.DMA,
        ],
    )
    def emb(idx_hbm, tpair_hbm, out_hbm, ix, pid, odd, P, S,
            gsem0, gsem1, wsem0, wsem1):
        wid = lax.axis_index("s") * NC + lax.axis_index("c")
        b0 = wid * CPT
        gsem = (gsem0, gsem1)
        wsem = (wsem0, wsem1)

        pltpu.sync_copy(idx_hbm.at[:, pl.ds(b0, CPT)], ix)

        def compute_pid(t, par):
            for g in range(CPT // 16):
                v = ix[t, pl.ds(g * 16, 16)]
                pid[par, pl.ds(g * 16, 16)] = jax.lax.shift_right_logical(v, 1)
                odd[par, pl.ds(g * 16, 16)] = jax.lax.shift_left(v & 1, 6)

        def fire_gather(par):
            pltpu.async_copy(tpair_hbm.at[pid.at[par]], P.at[par], gsem[par])

        def drain_gather(par):
            pltpu.make_async_copy(
                tpair_hbm.at[pid.at[par]], P.at[par], gsem[par]).wait()

        def fire_write(t, par):
            pltpu.async_copy(
                S.at[par], out_hbm.at[t, :, pl.ds(b0, CPT)], wsem[par])

        def drain_write(t, par):
            pltpu.make_async_copy(
                S.at[par], out_hbm.at[t, :, pl.ds(b0, CPT)], wsem[par]).wait()

        c16 = lax.iota(jnp.int32, 16)

        def transpose(par):
            Pp = P.at[par]
            ngrp = CPT // 16
            rows = [c16 + g * 16 for g in range(ngrp)]
            cols0 = [odd[par, pl.ds(g * 16, 16)] for g in range(ngrp)]

            def body(d, cols):
                for g in range(ngrp):
                    y = plsc.load_gather(Pp, [rows[g], cols[g]])
                    S[par, d, pl.ds(g * 16, 16)] = y
                return tuple(c + 1 for c in cols)

            lax.fori_loop(0, DIM, body, tuple(cols0))

        compute_pid(0, 0)
        fire_gather(0)

        def step(t, par, first, last):
            drain_gather(par)
            if not last:
                compute_pid(t + 1, 1 - par)
                fire_gather(1 - par)
            if not first:
                drain_write(t - 2, par)
            transpose(par)
            fire_write(t, par)

        step(0, 0, True, False)
        step(1, 1, True, False)

        def body(p, carry):
            step(2 * p, 0, False, False)
            step(2 * p + 1, 1, False, False)
            return carry

        lax.fori_loop(1, (T - 2) // 2, body, 0)

        step(T - 2, 0, False, False)
        step(T - 1, 1, False, True)
        drain_write(T - 2, 0)
        drain_write(T - 1, 1)

    return emb


def kernel(X, table):
    rows, cols = X.shape
    idxT = X.T.astype(jnp.int32)
    tpair = table.reshape(-1, 2 * DIM)
    out = _make_kernel(cols, rows, tpair.shape[0])(idxT, tpair)
    return out.transpose(2, 0, 1)


# speedup vs baseline: 1.3785x; 1.3785x over previous
import functools

import jax
import jax.numpy as jnp
from jax import lax
from jax.experimental import pallas as pl
from jax.experimental.pallas import tpu as pltpu
from jax.experimental.pallas import tpu_sc as plsc

DIM = 64
NC = 2
NS = 16
NW = NC * NS

CPT = 128


@functools.lru_cache(maxsize=None)
def _make_kernel(T: int, B: int, V2: int):
    assert B == NW * CPT and T % 2 == 0
    mesh = plsc.VectorSubcoreMesh(core_axis_name="c", subcore_axis_name="s")

    @functools.partial(
        pl.kernel,
        mesh=mesh,
        compiler_params=pltpu.CompilerParams(
            use_tc_tiling_on_sc=True, needs_layout_passes=False),
        out_type=jax.ShapeDtypeStruct((T, DIM, B), jnp.float32),
        scratch_types=[
            pltpu.VMEM((T, CPT), jnp.int32),
            pltpu.VMEM((2, CPT), jnp.int32),
            pltpu.VMEM((2, CPT), jnp.int32),
            pltpu.VMEM((2, CPT, 2 * DIM), jnp.float32),
            pltpu.VMEM((2, DIM, CPT), jnp.float32),
            pltpu.SemaphoreType.DMA,
            pltpu.SemaphoreType.DMA,
            pltpu.SemaphoreType.DMA,
            pltpu.SemaphoreType.DMA,
        ],
    )
    def emb(idx_hbm, tpair_hbm, out_hbm, ix, pid, odd, P, S,
            gsem0, gsem1, wsem0, wsem1):
        wid = lax.axis_index("s") * NC + lax.axis_index("c")
        b0 = wid * CPT
        gsem = (gsem0, gsem1)
        wsem = (wsem0, wsem1)

        pltpu.sync_copy(idx_hbm.at[:, pl.ds(b0, CPT)], ix)

        def compute_pid(t, par):
            for g in range(CPT // 16):
                v = ix[t, pl.ds(g * 16, 16)]
                pid[par, pl.ds(g * 16, 16)] = jax.lax.shift_right_logical(v, 1)
                odd[par, pl.ds(g * 16, 16)] = jax.lax.shift_left(v & 1, 6)

        def fire_gather(par):
            pltpu.async_copy(tpair_hbm.at[pid.at[par]], P.at[par], gsem[par])

        def drain_gather(par):
            pltpu.make_async_copy(
                tpair_hbm.at[pid.at[par]], P.at[par], gsem[par]).wait()

        def fire_write(t, par):
            pltpu.async_copy(
                S.at[par], out_hbm.at[t, :, pl.ds(b0, CPT)], wsem[par])

        def drain_write(t, par):
            pltpu.make_async_copy(
                S.at[par], out_hbm.at[t, :, pl.ds(b0, CPT)], wsem[par]).wait()

        c16 = lax.iota(jnp.int32, 16)

        def transpose(par):
            Pp = P.at[par]
            for g in range(CPT // 16):
                rowg = c16 + g * 16
                colg = odd[par, pl.ds(g * 16, 16)]

                def dblk(jb, colb):
                    ys = [plsc.load_gather(Pp, [rowg, colb + d])
                          for d in range(16)]
                    base = jb * 16
                    for d in range(16):
                        S[par, base + d, pl.ds(g * 16, 16)] = ys[d]
                    return colb + 16

                lax.fori_loop(0, DIM // 16, dblk, colg)

        compute_pid(0, 0)
        fire_gather(0)

        def step(t, par, first, last):
            drain_gather(par)
            if not last:
                compute_pid(t + 1, 1 - par)
                fire_gather(1 - par)
            if not first:
                drain_write(t - 2, par)
            transpose(par)
            fire_write(t, par)

        step(0, 0, True, False)
        step(1, 1, True, False)

        def body(p, carry):
            step(2 * p, 0, False, False)
            step(2 * p + 1, 1, False, False)
            return carry

        lax.fori_loop(1, (T - 2) // 2, body, 0)

        step(T - 2, 0, False, False)
        step(T - 1, 1, False, True)
        drain_write(T - 2, 0)
        drain_write(T - 1, 1)

    return emb


def kernel(X, table):
    rows, cols = X.shape
    idxT = X.T.astype(jnp.int32)
    tpair = table.reshape(-1, 2 * DIM)
    out = _make_kernel(cols, rows, tpair.shape[0])(idxT, tpair)
    return out.transpose(2, 0, 1)


# speedup vs baseline: 1.4888x; 1.0800x over previous
import functools

import jax
import jax.numpy as jnp
from jax import lax
from jax.experimental import pallas as pl
from jax.experimental.pallas import tpu as pltpu
from jax.experimental.pallas import tpu_sc as plsc

DIM = 64
NC = 2
NS = 16
NW = NC * NS

CHUNK = 512
NBUF = 3


@functools.lru_cache(maxsize=None)
def _make_kernel(B: int):
    assert B % (NW * CHUNK) == 0
    b_per_w = B // NW
    n_chunks = b_per_w // CHUNK
    mesh = plsc.VectorSubcoreMesh(core_axis_name="c", subcore_axis_name="s")

    steady_lo = 1
    steady_n = ((n_chunks - 2 - steady_lo) // NBUF) * NBUF
    steady_hi = steady_lo + steady_n

    @functools.partial(
        pl.kernel,
        mesh=mesh,
        compiler_params=pltpu.CompilerParams(use_tc_tiling_on_sc=False),
        out_type=jax.ShapeDtypeStruct((B, DIM), jnp.float32),
        scratch_types=[
            pltpu.VMEM((n_chunks, CHUNK), jnp.int32),
            pltpu.VMEM((NBUF, CHUNK, DIM), jnp.float32),
            pltpu.SemaphoreType.DMA,
            pltpu.SemaphoreType.DMA,
        ],
    )
    def emb(idx_hbm, table_hbm, out_hbm, idx_v, rows_v, gsem, wsem):
        wid = lax.axis_index("s") * NC + lax.axis_index("c")
        base = wid * b_per_w

        pltpu.sync_copy(idx_hbm.at[pl.ds(wid * n_chunks, n_chunks)], idx_v)

        def fire_gather(j, buf):
            pltpu.async_copy(table_hbm.at[idx_v.at[j]], rows_v.at[buf], gsem)

        def drain_gather(j, buf):
            pltpu.make_async_copy(
                table_hbm.at[idx_v.at[j]], rows_v.at[buf], gsem).wait()

        def fire_write(j, buf):
            pltpu.async_copy(
                rows_v.at[buf], out_hbm.at[pl.ds(base + j * CHUNK, CHUNK)], wsem)

        def drain_write(j, buf):
            pltpu.make_async_copy(
                rows_v.at[buf], out_hbm.at[pl.ds(base + j * CHUNK, CHUNK)],
                wsem).wait()

        fire_gather(0, 0)
        fire_gather(1, 1)
        drain_gather(0, 0)
        fire_gather(2, 2)
        fire_write(0, 0)

        def body(p, carry):
            for k in range(NBUF):
                j = steady_lo + p * NBUF + k
                b = (steady_lo + k) % NBUF
                drain_gather(j, b)
                drain_write(j - 1, (b + NBUF - 1) % NBUF)
                fire_gather(j + 2, (b + 2) % NBUF)
                fire_write(j, b)
            return carry

        lax.fori_loop(0, steady_n // NBUF, body, 0)

        for j in range(steady_hi, n_chunks):
            b = j % NBUF
            drain_gather(j, b)
            drain_write(j - 1, (b + NBUF - 1) % NBUF)
            if j + 2 < n_chunks:
                fire_gather(j + 2, (j + 2) % NBUF)
            fire_write(j, b)
        drain_write(n_chunks - 1, (n_chunks - 1) % NBUF)

    return emb


def kernel(X, table):
    rows, cols = X.shape
    idx = X.reshape(-1, CHUNK).astype(jnp.int32)
    out = _make_kernel(idx.size)(idx, table)
    return out.reshape(rows, cols, DIM)
